# baseline (speedup 1.0000x reference)
import jax
import jax.numpy as jnp
from jax.experimental import pallas as pl
from jax.experimental.pallas import tpu as pltpu

GROUP = 8
EPS = 1e-05


def _router_prep(w1_ref, b1_ref, b2_ref, w2_ref, w12t_ref, b12_ref):
    w12t = jax.lax.dot_general(
        w2_ref[...], w1_ref[...],
        (((0,), (1,)), ((), ())),
        preferred_element_type=jnp.float32)
    w12t_ref[...] = w12t
    b12_ref[...] = jnp.sum(w12t * b1_ref[...], axis=1,
                           keepdims=True) + b2_ref[...]


def _fused(x_ref, w12t_ref, b12_ref, w_ref, b_ref, out_ref,
           cache_ref, oh_ref, stats_ref, scale_ref, off_ref):
    p = pl.program_id(0)
    i = pl.program_id(1)
    n = pl.num_programs(1)
    hw = x_ref.shape[1]

    @pl.when((p == 0) & (i == 0))
    def _():
        stats_ref[...] = jnp.zeros_like(stats_ref)

    @pl.when(p == 0)
    def _():
        xb = x_ref[0]
        cache_ref[i] = xb
        lt = jnp.dot(w12t_ref[...], xb,
                     preferred_element_type=jnp.float32) + b12_ref[...]
        mx = jnp.max(lt, axis=0, keepdims=True)
        rowid = jax.lax.broadcasted_iota(jnp.int32, lt.shape, 0)
        idx = jnp.min(jnp.where(lt >= mx, rowid, GROUP), axis=0, keepdims=True)
        oh = (rowid == idx).astype(jnp.float32)
        oh_ref[i] = oh

        csum = jnp.sum(xb, axis=0, keepdims=True)
        cssq = jnp.sum(xb * xb, axis=0, keepdims=True)
        cnt_g = jnp.sum(oh, axis=1, keepdims=True)
        sum_g = jnp.sum(oh * csum, axis=1, keepdims=True)
        ssq_g = jnp.sum(oh * cssq, axis=1, keepdims=True)
        stats_ref[...] += jnp.concatenate([cnt_g, sum_g, ssq_g], axis=1)

    @pl.when((p == 1) & (i == 0))
    def _():
        cnt_rows = stats_ref[:, 0:1]
        total = cnt_rows * float(hw)
        s = stats_ref[:, 1:2]
        q = stats_ref[:, 2:3]
        mean = s / jnp.maximum(total, 1.0)
        sq = q - s * mean
        var = sq / jnp.maximum(total - 1.0, 1.0)
        rstd = jax.lax.rsqrt(var + EPS)
        for k in range(n):
            oh = oh_ref[k]
            rstd_c = jnp.sum(oh * rstd, axis=0, keepdims=True)
            mean_c = jnp.sum(oh * mean, axis=0, keepdims=True)
            sc = rstd_c * w_ref[...]
            scale_ref[k] = sc
            off_ref[k] = b_ref[...] - mean_c * sc

    @pl.when(p == 1)
    def _():
        out_ref[...] = cache_ref[i] * scale_ref[i] + off_ref[i]


def kernel(x, W1, b1, W2, b2, weight, bias):
    n, c, h, w = x.shape
    hw = h * w
    xt = jnp.transpose(x, (0, 2, 3, 1)).reshape(n, hw, c)

    w12t, b12 = pl.pallas_call(
        _router_prep,
        in_specs=[
            pl.BlockSpec((hw, hw), lambda: (0, 0)),
            pl.BlockSpec((1, hw), lambda: (0, 0)),
            pl.BlockSpec((GROUP, 1), lambda: (0, 0)),
            pl.BlockSpec((hw, GROUP), lambda: (0, 0)),
        ],
        out_specs=[
            pl.BlockSpec((GROUP, hw), lambda: (0, 0)),
            pl.BlockSpec((GROUP, 1), lambda: (0, 0)),
        ],
        out_shape=[
            jax.ShapeDtypeStruct((GROUP, hw), jnp.float32),
            jax.ShapeDtypeStruct((GROUP, 1), jnp.float32),
        ],
    )(W1, b1.reshape(1, hw), b2.reshape(GROUP, 1), W2)

    out2 = pl.pallas_call(
        _fused,
        grid=(2, n),
        in_specs=[
            pl.BlockSpec((1, hw, c), lambda p, i: (jnp.where(p == 0, i, n - 1), 0, 0)),
            pl.BlockSpec((GROUP, hw), lambda p, i: (0, 0)),
            pl.BlockSpec((GROUP, 1), lambda p, i: (0, 0)),
            pl.BlockSpec((1, c), lambda p, i: (0, 0)),
            pl.BlockSpec((1, c), lambda p, i: (0, 0)),
        ],
        out_specs=pl.BlockSpec((hw, c), lambda p, i: (jnp.where(p == 0, 0, i), 0)),
        out_shape=jax.ShapeDtypeStruct((n * hw, c), jnp.float32),
        scratch_shapes=[
            pltpu.VMEM((n, hw, c), jnp.float32),
            pltpu.VMEM((n, GROUP, c), jnp.float32),
            pltpu.VMEM((GROUP, 3), jnp.float32),
            pltpu.VMEM((n, 1, c), jnp.float32),
            pltpu.VMEM((n, 1, c), jnp.float32),
        ],
    )(xt, w12t, b12, weight.reshape(1, c), bias.reshape(1, c))

    return jnp.transpose(out2.reshape(n, h, w, c), (0, 3, 1, 2))
